# baseline (speedup 1.0000x reference)
import math

import jax
import jax.numpy as jnp
from jax.experimental import pallas as pl
from jax.experimental.pallas import tpu as pltpu

N_NODES = 100000
N_EDGES = 3200000
EPS = 1e-5
K0 = math.ceil(0.8 * N_NODES)
K1 = math.ceil(0.6 * K0)
K2 = math.ceil(0.4 * K1)


def _bn_eval(x, g, b):
    return x / jnp.sqrt(1.0 + EPS) * g + b


def _gcn_conv(x, src, dst, ew, W, b):
    n = x.shape[0]
    loop = jnp.arange(n, dtype=src.dtype)
    src2 = jnp.concatenate([src, loop])
    dst2 = jnp.concatenate([dst, loop])
    ew2 = jnp.concatenate([ew, jnp.ones((n,), x.dtype)])
    deg = jnp.zeros((n,), x.dtype).at[dst2].add(ew2)
    dinv = 1.0 / jnp.sqrt(deg)
    norm = dinv[src2] * ew2 * dinv[dst2]
    h = x @ W
    msg = h[src2] * norm[:, None]
    out = jnp.zeros_like(h).at[dst2].add(msg)
    return out + b


def _topk_pool(x, src, dst, ew, p_vec, k):
    n = x.shape[0]
    score = x @ p_vec / jnp.linalg.norm(p_vec)
    vals, perm = jax.lax.top_k(score, k)
    gate = jnp.tanh(vals)
    x_new = x[perm] * gate[:, None]
    mask = jnp.zeros((n,), dtype=bool).at[perm].set(True)
    newid = jnp.zeros((n,), dtype=src.dtype).at[perm].set(
        jnp.arange(k, dtype=src.dtype))
    valid = mask[src] & mask[dst]
    src_new = jnp.where(valid, newid[src], 0)
    dst_new = jnp.where(valid, newid[dst], 0)
    ew_new = ew * valid.astype(ew.dtype)
    return x_new, src_new, dst_new, ew_new


def _encoder(x, src, dst, ew, p, k):
    h = x @ p['mlp_W'] + p['mlp_b']
    h = _bn_eval(h, p['mlp_bn_g'], p['mlp_bn_b'])
    h = jax.nn.relu(h)
    h = _gcn_conv(h, src, dst, ew, p['conv_W'], p['conv_b'])
    h = _bn_eval(h, p['bn_g'], p['bn_b'])
    h = jax.nn.relu(h)
    return _topk_pool(h, src, dst, ew, p['pool_p'], k)


def _decode_kernel(xg_ref, w1_ref, b1_ref, g1_ref, bb1_ref, w2_ref, b2_ref,
                   out_ref):
    xg = xg_ref[...]
    h = xg @ w1_ref[...] + b1_ref[...]
    h = h / jnp.sqrt(1.0 + EPS) * g1_ref[...] + bb1_ref[...]
    h = jnp.maximum(h, 0.0)
    out_ref[...] = h @ w2_ref[...] + b2_ref[...]


def _decode(xg, d1p, d0p):
    return pl.pallas_call(
        _decode_kernel,
        out_shape=jax.ShapeDtypeStruct((1, d0p['W'].shape[1]), jnp.float32),
    )(xg, d1p['W'], d1p['b'][None, :], d1p['bn_g'][None, :],
      d1p['bn_b'][None, :], d0p['W'], d0p['b'][None, :])


def kernel(x, edge_index, params):
    src, dst = edge_index[0], edge_index[1]
    ew = jnp.ones((src.shape[0],), x.dtype)
    x0, s0, d0, w0 = _encoder(x, src, dst, ew, params['enc0'], K0)
    x1, s1, d1, w1 = _encoder(x0, s0, d0, w0, params['enc1'], K1)
    x2, s2, d2, w2 = _encoder(x1, s1, d1, w1, params['enc2'], K2)
    x3 = jnp.concatenate([x0, x1, x2], axis=0)
    xg = jnp.mean(x3, axis=0, keepdims=True)
    return _decode(xg, params['dec1'], params['dec0'])


# speedup vs baseline: 45.7817x; 45.7817x over previous
import functools
import math

import jax
import jax.numpy as jnp
from jax import lax
from jax.experimental import pallas as pl
from jax.experimental.pallas import tpu as pltpu
from jax.experimental.pallas import tpu_sc as plsc

N = 100000
E = 3200000
EPS = 1e-5
K0 = math.ceil(0.8 * N)
K1 = math.ceil(0.6 * K0)
K2 = math.ceil(0.4 * K1)

NC, NS, L = 2, 16, 16
NW = NC * NS
EPT = E // NW
CE = 2000
NBITS = 3200
C = 10240
NCHUNK = 10
AGG_ROWS = C * NCHUNK
G = 1024
NGB = G // 128

@functools.lru_cache(maxsize=None)
def _sc_kernels():
  _mesh = plsc.VectorSubcoreMesh(core_axis_name="c", subcore_axis_name="s",
                                 num_cores=NC, num_subcores=NS)
  _CP = pltpu.CompilerParams(needs_layout_passes=False,
                             use_tc_tiling_on_sc=False)

  @functools.partial(
    pl.kernel,
      out_type=(jax.ShapeDtypeStruct((E,), jnp.int32),
                jax.ShapeDtypeStruct((NW, N), jnp.float32)),
      mesh=_mesh, compiler_params=_CP,
      scratch_types=[
          pltpu.VMEM((NBITS,), jnp.int32),
          pltpu.VMEM((N,), jnp.float32),
          pltpu.VMEM((CE,), jnp.int32),
          pltpu.VMEM((CE,), jnp.int32),
          pltpu.VMEM((CE,), jnp.int32),
      ],
  )
  def _sc_remap_deg(src_hbm, dst_hbm, bits_hbm, dstn_hbm, degp_hbm,
                    bits_v, deg_v, src_v, dst_v, dn_v):
      wid = lax.axis_index("s") * NC + lax.axis_index("c")
      base = wid * EPT
      pltpu.sync_copy(bits_hbm, bits_v)

      def zero(i):
          deg_v[pl.ds(i * L, L)] = jnp.zeros((L,), jnp.float32)

      pl.loop(0, N // L)(zero)

      def chunk(ci):
          off = base + ci * CE
          pltpu.sync_copy(src_hbm.at[pl.ds(off, CE)], src_v)
          pltpu.sync_copy(dst_hbm.at[pl.ds(off, CE)], dst_v)

          def grp(g):
              s16 = src_v[pl.ds(g * L, L)]
              d16 = dst_v[pl.ds(g * L, L)]
              dc = jnp.maximum(d16, 0)
              ws = plsc.load_gather(bits_v, [jnp.right_shift(s16, 5)])
              wd = plsc.load_gather(bits_v, [jnp.right_shift(dc, 5)])
              bs = jnp.right_shift(ws, jnp.bitwise_and(s16, 31)) & 1
              bd = jnp.right_shift(wd, jnp.bitwise_and(dc, 31)) & 1
              ok = (bs & bd & jnp.where(d16 >= 0, 1, 0)) == 1
              dn_v[pl.ds(g * L, L)] = jnp.where(ok, d16, -1)
              plsc.addupdate_scatter(deg_v, [dc], jnp.ones((L,), jnp.float32),
                                     mask=ok)

          pl.loop(0, CE // L)(grp)
          pltpu.sync_copy(dn_v, dstn_hbm.at[pl.ds(off, CE)])

      pl.loop(0, EPT // CE)(chunk)
      pltpu.sync_copy(deg_v, degp_hbm.at[wid])


  @functools.partial(
      pl.kernel,
      out_type=jax.ShapeDtypeStruct((AGG_ROWS, 64), jnp.float32),
      mesh=_mesh, compiler_params=_CP,
      scratch_types=[
          pltpu.VMEM((CE,), jnp.int32),
          pltpu.VMEM((CE,), jnp.int32),
          pltpu.VMEM((G + L,), jnp.int32),
          pltpu.VMEM((G + L,), jnp.int32),
          pltpu.VMEM((NGB, 128), jnp.int32),
          pltpu.VMEM((NGB, 128), jnp.int32),
          pltpu.VMEM((G, 64), jnp.float32),
          pltpu.VMEM((128, 64), jnp.float32),
          pltpu.VMEM_SHARED((C + L, 64), jnp.float32),
          pltpu.SemaphoreType.DMA,
      ],
  )
  def _sc_msg(hh2_hbm, src_hbm, dstn_hbm, agg_hbm, src_v, dst_v, cg_v, cs_v,
              g2_v, s2_v, rows_v, zb_v, acc_sh, sem):
      cid = lax.axis_index("c")
      sid = lax.axis_index("s")
      base = sid * (E // NS)

      def zrow(r):
          def zcol(q):
              zb_v[r, pl.ds(q * L, L)] = jnp.zeros((L,), jnp.float32)

          pl.loop(0, 64 // L)(zcol)

      pl.loop(0, 128)(zrow)

      def flush_blocks(nb):
          def cp(b):
              def cpl(q):
                  g2_v[b, pl.ds(q * L, L)] = cg_v[pl.ds(b * 128 + q * L, L)]
                  s2_v[b, pl.ds(q * L, L)] = cs_v[pl.ds(b * 128 + q * L, L)]

              pl.loop(0, 128 // L)(cpl)

          pl.loop(0, nb)(cp)

          def dma(b):
              pltpu.async_copy(hh2_hbm.at[g2_v.at[b]],
                               rows_v.at[pl.ds(b * 128, 128)], sem).wait()

          pl.loop(0, nb)(dma)

          def sca(b):
              pltpu.sync_copy(rows_v.at[pl.ds(b * 128, 128)],
                              acc_sh.at[s2_v.at[b]], add=True)

          pl.loop(0, nb)(sca)

      for j in range(NCHUNK // NC):
          chunk_lo = (cid + NC * j) * C
          z0 = sid * ((C + L) // NS)

          def zchunk(t):
              pltpu.sync_copy(zb_v, acc_sh.at[pl.ds(z0 + t * 128, 128)])

          pl.loop(0, 5)(zchunk)
          pltpu.sync_copy(zb_v.at[pl.ds(0, 1)],
                          acc_sh.at[pl.ds(z0 + 640, 1)])
          plsc.subcore_barrier()

          def ce_chunk(ci, off):
              eoff = base + ci * CE
              pltpu.sync_copy(src_hbm.at[pl.ds(eoff, CE)], src_v)
              pltpu.sync_copy(dstn_hbm.at[pl.ds(eoff, CE)], dst_v)

              def grp(g, off):
                  s16 = src_v[pl.ds(g * L, L)]
                  d16 = dst_v[pl.ds(g * L, L)]
                  rel = d16 - chunk_lo
                  m = (rel >= 0) & (rel < C)
                  plsc.store_compressed(cg_v.at[pl.ds(off, L)], s16, mask=m)
                  plsc.store_compressed(cs_v.at[pl.ds(off, L)], rel, mask=m)
                  off2 = off + plsc.all_reduce_population_count(m)[0]

                  def do_flush():
                      flush_blocks(NGB)
                      cg_v[pl.ds(0, L)] = cg_v[pl.ds(G, L)]
                      cs_v[pl.ds(0, L)] = cs_v[pl.ds(G, L)]

                  pl.when(off2 >= G)(do_flush)
                  return jnp.where(off2 >= G, off2 - G, off2)

              return pl.loop(0, CE // L, init_carry=off)(grp)

          off = pl.loop(0, (E // NS) // CE, init_carry=0)(ce_chunk)

          nb = (off + 127) // 128

          def pad(p):
              cg_v[pl.ds(off + p * L, L)] = jnp.zeros((L,), jnp.int32)
              cs_v[pl.ds(off + p * L, L)] = jnp.full((L,), C, jnp.int32)

          pl.loop(0, (nb * 128 - off + L - 1) // L)(pad)
          flush_blocks(nb)
          plsc.subcore_barrier()

          w0 = sid * (C // NS)

          def wchunk(t):
              pltpu.sync_copy(acc_sh.at[pl.ds(w0 + t * 128, 128)],
                              agg_hbm.at[pl.ds(chunk_lo + w0 + t * 128, 128)])

          pl.loop(0, 5)(wchunk)
          plsc.subcore_barrier()


  return _sc_remap_deg, _sc_msg


def _bn(x, g, b):
    return x / jnp.sqrt(1.0 + EPS) * g + b


def _sortable(score):
    b = lax.bitcast_convert_type(score, jnp.int32)
    return b ^ (jnp.right_shift(b, 31) & jnp.int32(0x7FFFFFFF))


def _select_topk(score, alive, k):
    key = jnp.where(alive, _sortable(score), jnp.int32(-2147483648))
    ukey = lax.bitcast_convert_type(key, jnp.uint32) ^ jnp.uint32(0x80000000)

    def bit_step(i, tu):
        cand = tu | (jnp.uint32(1) << (jnp.uint32(31) - i.astype(jnp.uint32)))
        cnt = jnp.sum((ukey >= cand).astype(jnp.int32))
        return jnp.where(cnt >= k, cand, tu)

    tu = lax.fori_loop(0, 32, bit_step, jnp.uint32(0))
    gt = ukey > tu
    eq = ukey == tu
    need = k - jnp.sum(gt.astype(jnp.int32))
    cs = jnp.cumsum(eq.astype(jnp.int32))
    return gt | (eq & (cs <= need))


def _pack_bits(keep):
    pad = jnp.zeros((NBITS * 32 - N,), dtype=bool)
    kp = jnp.concatenate([keep, pad]).reshape(NBITS, 32).astype(jnp.uint32)
    w = jnp.sum(kp << jnp.arange(32, dtype=jnp.uint32), axis=1)
    return lax.bitcast_convert_type(w, jnp.int32)


def _decode_kernel(xg_ref, w1_ref, b1_ref, g1_ref, bb1_ref, w2_ref, b2_ref,
                   out_ref):
    xg = xg_ref[...]
    h = xg @ w1_ref[...] + b1_ref[...]
    h = h / jnp.sqrt(1.0 + EPS) * g1_ref[...] + bb1_ref[...]
    h = jnp.maximum(h, 0.0)
    out_ref[...] = h @ w2_ref[...] + b2_ref[...]


def _level(x_in, src, dst_cur, bits, alive, p, k):
    _sc_remap_deg, _sc_msg = _sc_kernels()
    dstn, degp = _sc_remap_deg(src, dst_cur, bits)
    deg = 1.0 + jnp.sum(degp, axis=0)
    dinv = lax.rsqrt(deg)
    h = jax.nn.relu(_bn(x_in @ p['mlp_W'] + p['mlp_b'],
                        p['mlp_bn_g'], p['mlp_bn_b']))
    hh2 = (h @ p['conv_W']) * dinv[:, None]
    agg = _sc_msg(hh2, src, dstn)[:N]
    x3 = jax.nn.relu(_bn((agg + hh2) * dinv[:, None] + p['conv_b'],
                         p['bn_g'], p['bn_b']))
    score = x3 @ p['pool_p'] / jnp.linalg.norm(p['pool_p'])
    keep = _select_topk(score, alive, k)
    gate = jnp.tanh(score) * keep.astype(jnp.float32)
    xout = x3 * gate[:, None]
    return xout, dstn, _pack_bits(keep), keep


def kernel(x, edge_index, params):
    src, dst = edge_index[0], edge_index[1]
    ones_bits = jnp.full((NBITS,), -1, jnp.int32)
    all_alive = jnp.ones((N,), dtype=bool)

    x0, dst0, bits1, keep0 = _level(x, src, dst, ones_bits, all_alive,
                                    params['enc0'], K0)
    x1, dst1, bits2, keep1 = _level(x0, src, dst0, bits1, keep0,
                                    params['enc1'], K1)
    x2, _, _, _ = _level(x1, src, dst1, bits2, keep1, params['enc2'], K2)

    tot = jnp.sum(x0, axis=0) + jnp.sum(x1, axis=0) + jnp.sum(x2, axis=0)
    xg = (tot / float(K0 + K1 + K2))[None, :]

    d1p, d0p = params['dec1'], params['dec0']
    return pl.pallas_call(
        _decode_kernel,
        out_shape=jax.ShapeDtypeStruct((1, d0p['W'].shape[1]), jnp.float32),
    )(xg, d1p['W'], d1p['b'][None, :], d1p['bn_g'][None, :],
      d1p['bn_b'][None, :], d0p['W'], d0p['b'][None, :])
